# baseline (speedup 1.0000x reference)
import functools

import jax
import jax.numpy as jnp
from jax import lax
from jax.experimental import pallas as pl
from jax.experimental.pallas import tpu as pltpu
from jax.experimental.pallas import tpu_sc as plsc

NC = 2
NS = 16
NW = NC * NS
NBUF = 8


def kernel(idx, weight):
    b, h = idx.shape
    v, d = weight.shape
    bpw = b // NW
    mesh = plsc.VectorSubcoreMesh(core_axis_name="c", subcore_axis_name="s")

    @functools.partial(
        pl.kernel,
        mesh=mesh,
        compiler_params=pltpu.CompilerParams(use_tc_tiling_on_sc=False),
        out_type=jax.ShapeDtypeStruct((b, h, d), jnp.float32),
        scratch_types=[
            pltpu.VMEM((bpw, h), jnp.int32),
            pltpu.VMEM((NBUF, h, d), jnp.float32),
            pltpu.SemaphoreType.DMA,
            pltpu.SemaphoreType.DMA,
        ],
    )
    def k(idx_hbm, tbl_hbm, out_hbm, idx_v, rows_v, gsem, ssem):
        wid = lax.axis_index("s") * NC + lax.axis_index("c")
        base = wid * bpw
        pltpu.sync_copy(idx_hbm.at[pl.ds(base, bpw)], idx_v)

        def gather(g, buf):
            pltpu.async_copy(tbl_hbm.at[idx_v.at[g]], rows_v.at[buf], gsem)

        for i in range(NBUF):
            gather(i, i)

        def body(j, carry):
            buf = lax.rem(j, NBUF)
            pltpu.make_async_copy(
                tbl_hbm.at[idx_v.at[j]], rows_v.at[buf], gsem
            ).wait()
            pltpu.async_copy(rows_v.at[buf], out_hbm.at[base + j], ssem)

            @pl.when((j >= 1) & (j <= bpw - NBUF))
            def _():
                pltpu.make_async_copy(
                    rows_v.at[buf], out_hbm.at[base], ssem
                ).wait()
                g = j - 1 + NBUF
                gather(g, lax.rem(g, NBUF))

            return carry

        lax.fori_loop(0, bpw, body, 0)

        for _ in range(NBUF):
            pltpu.make_async_copy(
                rows_v.at[0], out_hbm.at[base], ssem
            ).wait()

    return k(idx, weight)
